# baseline (speedup 1.0000x reference)
import functools

import jax
import jax.numpy as jnp
from jax import lax
from jax.experimental import pallas as pl
from jax.experimental.pallas import tpu as pltpu
from jax.experimental.pallas import tpu_sc as plsc

_E = 8
_K = 2
_D = 1024
_FF = 4096
_T = 2048

_NC = 2
_NS = 16
_NW = _NC * _NS
_L = 16

_TB = 128
_TB_LOG2 = 7
_NT = _T * _K // _TB + _E
_SP = _NT * _TB
_NTE = 48
_APW = _T * _K // _NW
_TPW = _T // _NW


def _gating_body(x_ref, gw_ref, gb_ref, code_ref, w1n_ref, loss_ref):
    x = x_ref[...]
    logits = jnp.dot(x, gw_ref[...], preferred_element_type=jnp.float32)
    logits = logits + gb_ref[...]
    m = jnp.max(logits, axis=-1, keepdims=True)
    ex = jnp.exp(logits - m)
    probs = ex / jnp.sum(ex, axis=-1, keepdims=True)
    eidx = lax.broadcasted_iota(jnp.int32, probs.shape, 1)
    m1 = jnp.max(probs, axis=-1, keepdims=True)
    i1 = jnp.min(jnp.where(probs == m1, eidx, _E), axis=-1, keepdims=True)
    probs2 = jnp.where(eidx == i1, -1.0, probs)
    m2 = jnp.max(probs2, axis=-1, keepdims=True)
    i2 = jnp.min(jnp.where(probs2 == m2, eidx, _E), axis=-1, keepdims=True)
    code_ref[...] = i1 * _E + i2
    w1n_ref[...] = m1 / (m1 + m2)
    oh = ((eidx == i1) | (eidx == i2)).astype(jnp.float32)
    imp = jnp.sum(probs, axis=0, keepdims=True)
    imp = imp / jnp.sum(imp)
    load = jnp.sum(oh, axis=0, keepdims=True) / (_T * _K)
    loss_ref[...] = _E * jnp.sum(imp * load, axis=(0, 1), keepdims=True)


_SC_MESH = plsc.VectorSubcoreMesh(
    core_axis_name="c", subcore_axis_name="s", num_cores=_NC, num_subcores=_NS
)


@functools.partial(
    pl.kernel,
    out_type=(
        jax.ShapeDtypeStruct((_SP, _D), jnp.float32),
        jax.ShapeDtypeStruct((_T * _K,), jnp.int32),
        jax.ShapeDtypeStruct((_NTE,), jnp.int32),
    ),
    mesh=_SC_MESH,
    scratch_types=[
        pltpu.VMEM((_T,), jnp.int32),
        pltpu.VMEM((_APW,), jnp.int32),
        pltpu.VMEM((_L,), jnp.int32),
        pltpu.VMEM((_L,), jnp.int32),
        pltpu.VMEM((_L, _D), jnp.float32),
        pltpu.VMEM((_NTE,), jnp.int32),
        pltpu.SemaphoreType.DMA,
        pltpu.SemaphoreType.DMA,
    ],
    compiler_params=pltpu.CompilerParams(needs_layout_passes=False),
)
def _route_kernel(code_hbm, x_hbm, dx_hbm, slots_hbm, te_hbm,
                  codes_v, slots_v, gidx_v, sidx_v, rows_v, te_v,
                  gsem, ssem):
    core = lax.axis_index("c")
    sub = lax.axis_index("s")
    wid = sub * _NC + core
    lane = lax.broadcasted_iota(jnp.int32, (_L,), 0)

    pltpu.sync_copy(code_hbm, codes_v)

    def count_step(j, c):
        v = codes_v[pl.ds(j * _L, _L)]
        e1 = lax.shift_right_logical(v, 3)
        e2 = lax.bitwise_and(v, 7)
        for e in range(_E):
            cnt = jnp.sum(jnp.where(e1 == e, 1, 0) + jnp.where(e2 == e, 1, 0))
            c = jnp.where(lane == e, c + cnt, c)
        return c

    prefix = lax.fori_loop(0, wid * (_TPW // _L), count_step,
                           jnp.zeros((_L,), jnp.int32))
    total = lax.fori_loop(wid * (_TPW // _L), _T // _L, count_step, prefix)

    pc = lax.shift_left(
        lax.shift_right_logical(total + (_TB - 1), _TB_LOG2), _TB_LOG2)
    cum = plsc.cumsum(pc)
    off = cum - pc

    @pl.when(wid == 0)
    def _():
        for half in range(_NTE // _L):
            tile_ids = (lane + half * _L) * _TB
            te = jnp.zeros((_L,), jnp.int32)
            for e in range(_E):
                bound = jnp.sum(jnp.where(lane == e, cum, 0))
                te = te + jnp.where(tile_ids >= bound, 1, 0)
            te_v[pl.ds(half * _L, _L)] = jnp.minimum(te, _E - 1)
        pltpu.sync_copy(te_v, te_hbm)

    opr = off + prefix
    for j in range(_APW // _L):
        a_vec = wid * _APW + j * _L + lane
        toks = lax.shift_right_logical(a_vec, 1)
        kk = lax.bitwise_and(a_vec, 1)
        cvec = plsc.load_gather(codes_v, [toks])
        evec = jnp.where(kk == 0,
                         lax.shift_right_logical(cvec, 3),
                         lax.bitwise_and(cvec, 7))
        slot = jnp.zeros((_L,), jnp.int32)
        newopr = opr
        for e in range(_E):
            msk = evec == e
            cs = plsc.cumsum(jnp.where(msk, 1, 0))
            base_e = jnp.sum(jnp.where(lane == e, opr, 0))
            slot = jnp.where(msk, base_e + cs - 1, slot)
            cnt = jnp.sum(jnp.where(msk, 1, 0))
            newopr = jnp.where(lane == e, newopr + cnt, newopr)
        opr = newopr
        slots_v[pl.ds(j * _L, _L)] = slot
    pltpu.sync_copy(slots_v, slots_hbm.at[pl.ds(wid * _APW, _APW)])

    for j in range(_APW // _L):
        a_vec = wid * _APW + j * _L + lane
        gidx_v[...] = lax.shift_right_logical(a_vec, 1)
        sidx_v[...] = slots_v[pl.ds(j * _L, _L)]
        pltpu.async_copy(x_hbm.at[gidx_v], rows_v, gsem).wait()
        pltpu.async_copy(rows_v, dx_hbm.at[sidx_v], ssem).wait()


def _ffn_body(te_ref, dx_ref, w1_ref, b1_ref, w2_ref, b2_ref, yin_ref, y_ref):
    f = pl.program_id(0)
    h = jnp.dot(dx_ref[...], w1_ref[0], preferred_element_type=jnp.float32)
    h = jnp.maximum(h + b1_ref[0], 0.0)
    part = jnp.dot(h, w2_ref[0], preferred_element_type=jnp.float32)

    @pl.when(f == 0)
    def _():
        y_ref[...] = part + b2_ref[0]

    @pl.when(f == 1)
    def _():
        y_ref[...] = yin_ref[...] + part


@functools.partial(
    pl.kernel,
    out_type=jax.ShapeDtypeStruct((_T, _D), jnp.float32),
    mesh=_SC_MESH,
    scratch_types=[
        pltpu.VMEM((_APW,), jnp.int32),
        pltpu.VMEM((_TPW,), jnp.float32),
        pltpu.VMEM((_L,), jnp.int32),
        pltpu.VMEM((_L, _D), jnp.float32),
        pltpu.VMEM((_L // 2, _D), jnp.float32),
        pltpu.SemaphoreType.DMA,
    ],
    compiler_params=pltpu.CompilerParams(needs_layout_passes=False),
)
def _combine_kernel(y_hbm, slots_hbm, w1n_hbm, out_hbm,
                    slots_v, w_v, sidx_v, rows_v, ov, sem):
    core = lax.axis_index("c")
    sub = lax.axis_index("s")
    wid = sub * _NC + core

    pltpu.sync_copy(slots_hbm.at[pl.ds(wid * _APW, _APW)], slots_v)
    pltpu.sync_copy(w1n_hbm.at[pl.ds(wid * _TPW, _TPW)], w_v)

    ntok = _L // 2
    for g in range(_TPW // ntok):
        sidx_v[...] = slots_v[pl.ds(g * _L, _L)]
        pltpu.async_copy(y_hbm.at[sidx_v], rows_v, sem).wait()
        lane = lax.broadcasted_iota(jnp.int32, (_L,), 0)
        wchunk = w_v[pl.ds((g // 2) * _L, _L)]
        wsplat = [
            jnp.sum(jnp.where(lane == (g % 2) * ntok + i, wchunk, 0.0))
            for i in range(ntok)
        ]

        def col_step(c, carry):
            for i in range(ntok):
                r0 = rows_v[2 * i, pl.ds(c * _L, _L)]
                r1 = rows_v[2 * i + 1, pl.ds(c * _L, _L)]
                ov[i, pl.ds(c * _L, _L)] = wsplat[i] * (r0 - r1) + r1
            return carry

        lax.fori_loop(0, _D // _L, col_step, jnp.int32(0))
        pltpu.sync_copy(ov, out_hbm.at[pl.ds(wid * _TPW + g * ntok, ntok)])


def kernel(x, gate_W, gate_b, W1, b1, W2, b2):
    code, w1n, loss = pl.pallas_call(
        _gating_body,
        out_shape=(
            jax.ShapeDtypeStruct((_T, 1), jnp.int32),
            jax.ShapeDtypeStruct((_T, 1), jnp.float32),
            jax.ShapeDtypeStruct((1, 1), jnp.float32),
        ),
    )(x, gate_W, gate_b.reshape(1, _E))

    dx, slots, te = _route_kernel(code.reshape(_T), x)

    grid_spec = pltpu.PrefetchScalarGridSpec(
        num_scalar_prefetch=1,
        grid=(2, _NT),
        in_specs=[
            pl.BlockSpec((_TB, _D), lambda f, i, te: (i, 0)),
            pl.BlockSpec((1, _D, _FF // 2), lambda f, i, te: (te[i], 0, f)),
            pl.BlockSpec((1, 1, _FF // 2), lambda f, i, te: (te[i], 0, f)),
            pl.BlockSpec((1, _FF // 2, _D), lambda f, i, te: (te[i], f, 0)),
            pl.BlockSpec((1, 1, _D), lambda f, i, te: (te[i], 0, 0)),
            pl.BlockSpec((_TB, _D), lambda f, i, te: (i, 0)),
        ],
        out_specs=pl.BlockSpec((_TB, _D), lambda f, i, te: (i, 0)),
    )
    y = pl.pallas_call(
        _ffn_body,
        grid_spec=grid_spec,
        out_shape=jax.ShapeDtypeStruct((_SP, _D), jnp.float32),
        input_output_aliases={6: 0},
        compiler_params=pltpu.CompilerParams(
            dimension_semantics=("arbitrary", "arbitrary"),
        ),
    )(
        te,
        dx,
        W1,
        b1.reshape(_E, 1, _FF),
        W2,
        b2.reshape(_E, 1, _D),
        jnp.zeros((_SP, _D), jnp.float32),
    )

    out = _combine_kernel(y, slots, w1n.reshape(_T))
    return out, loss[0, 0]


# speedup vs baseline: 1.0150x; 1.0150x over previous
import functools

import jax
import jax.numpy as jnp
from jax import lax
from jax.experimental import pallas as pl
from jax.experimental.pallas import tpu as pltpu
from jax.experimental.pallas import tpu_sc as plsc

_E = 8
_K = 2
_D = 1024
_FF = 4096
_T = 2048

_NC = 2
_NS = 16
_NW = _NC * _NS
_L = 16

_TB = 256
_TB_LOG2 = 8
_NT = _T * _K // _TB + _E
_SP = _NT * _TB
_NTE = 48
_APW = _T * _K // _NW
_TPW = _T // _NW


def _gating_body(x_ref, gw_ref, gb_ref, code_ref, w1n_ref, loss_ref):
    x = x_ref[...]
    logits = jnp.dot(x, gw_ref[...], preferred_element_type=jnp.float32)
    logits = logits + gb_ref[...]
    m = jnp.max(logits, axis=-1, keepdims=True)
    ex = jnp.exp(logits - m)
    probs = ex / jnp.sum(ex, axis=-1, keepdims=True)
    eidx = lax.broadcasted_iota(jnp.int32, probs.shape, 1)
    m1 = jnp.max(probs, axis=-1, keepdims=True)
    i1 = jnp.min(jnp.where(probs == m1, eidx, _E), axis=-1, keepdims=True)
    probs2 = jnp.where(eidx == i1, -1.0, probs)
    m2 = jnp.max(probs2, axis=-1, keepdims=True)
    i2 = jnp.min(jnp.where(probs2 == m2, eidx, _E), axis=-1, keepdims=True)
    code_ref[...] = i1 * _E + i2
    w1n_ref[...] = m1 / (m1 + m2)
    oh = ((eidx == i1) | (eidx == i2)).astype(jnp.float32)
    imp = jnp.sum(probs, axis=0, keepdims=True)
    imp = imp / jnp.sum(imp)
    load = jnp.sum(oh, axis=0, keepdims=True) / (_T * _K)
    loss_ref[...] = _E * jnp.sum(imp * load, axis=(0, 1), keepdims=True)


_SC_MESH = plsc.VectorSubcoreMesh(
    core_axis_name="c", subcore_axis_name="s", num_cores=_NC, num_subcores=_NS
)


@functools.partial(
    pl.kernel,
    out_type=(
        jax.ShapeDtypeStruct((_SP, _D), jnp.float32),
        jax.ShapeDtypeStruct((_T * _K,), jnp.int32),
        jax.ShapeDtypeStruct((_NTE,), jnp.int32),
    ),
    mesh=_SC_MESH,
    scratch_types=[
        pltpu.VMEM((_T,), jnp.int32),
        pltpu.VMEM((_APW,), jnp.int32),
        pltpu.VMEM((_L,), jnp.int32),
        pltpu.VMEM((_L,), jnp.int32),
        pltpu.VMEM((_L, _D), jnp.float32),
        pltpu.VMEM((_NTE,), jnp.int32),
        pltpu.SemaphoreType.DMA,
        pltpu.SemaphoreType.DMA,
    ],
    compiler_params=pltpu.CompilerParams(needs_layout_passes=False),
)
def _route_kernel(code_hbm, x_hbm, dx_hbm, slots_hbm, te_hbm,
                  codes_v, slots_v, gidx_v, sidx_v, rows_v, te_v,
                  gsem, ssem):
    core = lax.axis_index("c")
    sub = lax.axis_index("s")
    wid = sub * _NC + core
    lane = lax.broadcasted_iota(jnp.int32, (_L,), 0)

    pltpu.sync_copy(code_hbm, codes_v)

    def count_step(j, c):
        v = codes_v[pl.ds(j * _L, _L)]
        e1 = lax.shift_right_logical(v, 3)
        e2 = lax.bitwise_and(v, 7)
        for e in range(_E):
            cnt = jnp.sum(jnp.where(e1 == e, 1, 0) + jnp.where(e2 == e, 1, 0))
            c = jnp.where(lane == e, c + cnt, c)
        return c

    prefix = lax.fori_loop(0, wid * (_TPW // _L), count_step,
                           jnp.zeros((_L,), jnp.int32))
    total = lax.fori_loop(wid * (_TPW // _L), _T // _L, count_step, prefix)

    pc = lax.shift_left(
        lax.shift_right_logical(total + (_TB - 1), _TB_LOG2), _TB_LOG2)
    cum = plsc.cumsum(pc)
    off = cum - pc

    @pl.when(wid == 0)
    def _():
        for half in range(_NTE // _L):
            tile_ids = (lane + half * _L) * _TB
            te = jnp.zeros((_L,), jnp.int32)
            for e in range(_E):
                bound = jnp.sum(jnp.where(lane == e, cum, 0))
                te = te + jnp.where(tile_ids >= bound, 1, 0)
            te_v[pl.ds(half * _L, _L)] = jnp.minimum(te, _E - 1)
        pltpu.sync_copy(te_v, te_hbm)

    opr = off + prefix
    for j in range(_APW // _L):
        a_vec = wid * _APW + j * _L + lane
        toks = lax.shift_right_logical(a_vec, 1)
        kk = lax.bitwise_and(a_vec, 1)
        cvec = plsc.load_gather(codes_v, [toks])
        evec = jnp.where(kk == 0,
                         lax.shift_right_logical(cvec, 3),
                         lax.bitwise_and(cvec, 7))
        slot = jnp.zeros((_L,), jnp.int32)
        newopr = opr
        for e in range(_E):
            msk = evec == e
            cs = plsc.cumsum(jnp.where(msk, 1, 0))
            base_e = jnp.sum(jnp.where(lane == e, opr, 0))
            slot = jnp.where(msk, base_e + cs - 1, slot)
            cnt = jnp.sum(jnp.where(msk, 1, 0))
            newopr = jnp.where(lane == e, newopr + cnt, newopr)
        opr = newopr
        slots_v[pl.ds(j * _L, _L)] = slot
    pltpu.sync_copy(slots_v, slots_hbm.at[pl.ds(wid * _APW, _APW)])

    for j in range(_APW // _L):
        a_vec = wid * _APW + j * _L + lane
        gidx_v[...] = lax.shift_right_logical(a_vec, 1)
        sidx_v[...] = slots_v[pl.ds(j * _L, _L)]
        pltpu.async_copy(x_hbm.at[gidx_v], rows_v, gsem).wait()
        pltpu.async_copy(rows_v, dx_hbm.at[sidx_v], ssem).wait()


def _ffn_body(te_ref, dx_ref, w1_ref, b1_ref, w2_ref, b2_ref, yin_ref, y_ref):
    f = pl.program_id(0)
    h = jnp.dot(dx_ref[...], w1_ref[0], preferred_element_type=jnp.float32)
    h = jnp.maximum(h + b1_ref[0], 0.0)
    part = jnp.dot(h, w2_ref[0], preferred_element_type=jnp.float32)

    @pl.when(f == 0)
    def _():
        y_ref[...] = part + b2_ref[0]

    @pl.when(f == 1)
    def _():
        y_ref[...] = yin_ref[...] + part


@functools.partial(
    pl.kernel,
    out_type=jax.ShapeDtypeStruct((_T, _D), jnp.float32),
    mesh=_SC_MESH,
    scratch_types=[
        pltpu.VMEM((_APW,), jnp.int32),
        pltpu.VMEM((_TPW,), jnp.float32),
        pltpu.VMEM((_L,), jnp.int32),
        pltpu.VMEM((_L, _D), jnp.float32),
        pltpu.VMEM((_L // 2, _D), jnp.float32),
        pltpu.SemaphoreType.DMA,
    ],
    compiler_params=pltpu.CompilerParams(needs_layout_passes=False),
)
def _combine_kernel(y_hbm, slots_hbm, w1n_hbm, out_hbm,
                    slots_v, w_v, sidx_v, rows_v, ov, sem):
    core = lax.axis_index("c")
    sub = lax.axis_index("s")
    wid = sub * _NC + core

    pltpu.sync_copy(slots_hbm.at[pl.ds(wid * _APW, _APW)], slots_v)
    pltpu.sync_copy(w1n_hbm.at[pl.ds(wid * _TPW, _TPW)], w_v)

    ntok = _L // 2
    for g in range(_TPW // ntok):
        sidx_v[...] = slots_v[pl.ds(g * _L, _L)]
        pltpu.async_copy(y_hbm.at[sidx_v], rows_v, sem).wait()
        lane = lax.broadcasted_iota(jnp.int32, (_L,), 0)
        wchunk = w_v[pl.ds((g // 2) * _L, _L)]
        wsplat = [
            jnp.sum(jnp.where(lane == (g % 2) * ntok + i, wchunk, 0.0))
            for i in range(ntok)
        ]

        def col_step(c, carry):
            for i in range(ntok):
                r0 = rows_v[2 * i, pl.ds(c * _L, _L)]
                r1 = rows_v[2 * i + 1, pl.ds(c * _L, _L)]
                ov[i, pl.ds(c * _L, _L)] = wsplat[i] * (r0 - r1) + r1
            return carry

        lax.fori_loop(0, _D // _L, col_step, jnp.int32(0))
        pltpu.sync_copy(ov, out_hbm.at[pl.ds(wid * _TPW + g * ntok, ntok)])


def kernel(x, gate_W, gate_b, W1, b1, W2, b2):
    code, w1n, loss = pl.pallas_call(
        _gating_body,
        out_shape=(
            jax.ShapeDtypeStruct((_T, 1), jnp.int32),
            jax.ShapeDtypeStruct((_T, 1), jnp.float32),
            jax.ShapeDtypeStruct((1, 1), jnp.float32),
        ),
    )(x, gate_W, gate_b.reshape(1, _E))

    dx, slots, te = _route_kernel(code.reshape(_T), x)

    grid_spec = pltpu.PrefetchScalarGridSpec(
        num_scalar_prefetch=1,
        grid=(2, _NT),
        in_specs=[
            pl.BlockSpec((_TB, _D), lambda f, i, te: (i, 0)),
            pl.BlockSpec((1, _D, _FF // 2), lambda f, i, te: (te[i], 0, f)),
            pl.BlockSpec((1, 1, _FF // 2), lambda f, i, te: (te[i], 0, f)),
            pl.BlockSpec((1, _FF // 2, _D), lambda f, i, te: (te[i], f, 0)),
            pl.BlockSpec((1, 1, _D), lambda f, i, te: (te[i], 0, 0)),
            pl.BlockSpec((_TB, _D), lambda f, i, te: (i, 0)),
        ],
        out_specs=pl.BlockSpec((_TB, _D), lambda f, i, te: (i, 0)),
    )
    y = pl.pallas_call(
        _ffn_body,
        grid_spec=grid_spec,
        out_shape=jax.ShapeDtypeStruct((_SP, _D), jnp.float32),
        input_output_aliases={6: 0},
        compiler_params=pltpu.CompilerParams(
            dimension_semantics=("arbitrary", "arbitrary"),
        ),
    )(
        te,
        dx,
        W1,
        b1.reshape(_E, 1, _FF),
        W2,
        b2.reshape(_E, 1, _D),
        jnp.zeros((_SP, _D), jnp.float32),
    )

    out = _combine_kernel(y, slots, w1n.reshape(_T))
    return out, loss[0, 0]


# speedup vs baseline: 1.1461x; 1.1291x over previous
import functools

import jax
import jax.numpy as jnp
from jax import lax
from jax.experimental import pallas as pl
from jax.experimental.pallas import tpu as pltpu
from jax.experimental.pallas import tpu_sc as plsc

_E = 8
_K = 2
_D = 1024
_FF = 4096
_T = 2048

_NC = 2
_NS = 16
_NW = _NC * _NS
_L = 16

_TB = 256
_TB_LOG2 = 8
_NT = _T * _K // _TB + _E
_SP = _NT * _TB
_NTE = 48
_APW = _T * _K // _NW
_TPW = _T // _NW


def _gating_body(x_ref, gw_ref, gb_ref, code_ref, w1n_ref, loss_ref):
    x = x_ref[...]
    logits = jnp.dot(x, gw_ref[...], preferred_element_type=jnp.float32)
    logits = logits + gb_ref[...]
    m = jnp.max(logits, axis=-1, keepdims=True)
    ex = jnp.exp(logits - m)
    probs = ex / jnp.sum(ex, axis=-1, keepdims=True)
    eidx = lax.broadcasted_iota(jnp.int32, probs.shape, 1)
    m1 = jnp.max(probs, axis=-1, keepdims=True)
    i1 = jnp.min(jnp.where(probs == m1, eidx, _E), axis=-1, keepdims=True)
    probs2 = jnp.where(eidx == i1, -1.0, probs)
    m2 = jnp.max(probs2, axis=-1, keepdims=True)
    i2 = jnp.min(jnp.where(probs2 == m2, eidx, _E), axis=-1, keepdims=True)
    code_ref[...] = i1 * _E + i2
    w1n_ref[...] = m1 / (m1 + m2)
    oh = ((eidx == i1) | (eidx == i2)).astype(jnp.float32)
    imp = jnp.sum(probs, axis=0, keepdims=True)
    imp = imp / jnp.sum(imp)
    load = jnp.sum(oh, axis=0, keepdims=True) / (_T * _K)
    loss_ref[...] = _E * jnp.sum(imp * load, axis=(0, 1), keepdims=True)


_SC_MESH = plsc.VectorSubcoreMesh(
    core_axis_name="c", subcore_axis_name="s", num_cores=_NC, num_subcores=_NS
)


@functools.partial(
    pl.kernel,
    out_type=(
        jax.ShapeDtypeStruct((_SP, _D), jnp.float32),
        jax.ShapeDtypeStruct((_T * _K,), jnp.int32),
        jax.ShapeDtypeStruct((_NTE,), jnp.int32),
        jax.ShapeDtypeStruct((_L,), jnp.int32),
    ),
    mesh=_SC_MESH,
    scratch_types=[
        pltpu.VMEM((_T,), jnp.int32),
        pltpu.VMEM((_APW,), jnp.int32),
        pltpu.VMEM((2, _L), jnp.int32),
        pltpu.VMEM((2, _L), jnp.int32),
        pltpu.VMEM((2, _L, _D), jnp.float32),
        pltpu.VMEM((_NTE,), jnp.int32),
        pltpu.VMEM((_L,), jnp.int32),
        pltpu.SemaphoreType.DMA,
        pltpu.SemaphoreType.DMA,
        pltpu.SemaphoreType.DMA,
    ],
    compiler_params=pltpu.CompilerParams(needs_layout_passes=False),
)
def _route_kernel(code_hbm, x_hbm, dx_hbm, slots_hbm, te_hbm, nt_hbm,
                  codes_v, slots_v, gidx_v, sidx_v, rows_v, te_v, nt_v,
                  gsem, ssem_a, ssem_b):
    core = lax.axis_index("c")
    sub = lax.axis_index("s")
    wid = sub * _NC + core
    lane = lax.broadcasted_iota(jnp.int32, (_L,), 0)

    pltpu.sync_copy(code_hbm, codes_v)

    def count_step(j, c):
        v = codes_v[pl.ds(j * _L, _L)]
        e1 = lax.shift_right_logical(v, 3)
        e2 = lax.bitwise_and(v, 7)
        for e in range(_E):
            cnt = jnp.sum(jnp.where(e1 == e, 1, 0) + jnp.where(e2 == e, 1, 0))
            c = jnp.where(lane == e, c + cnt, c)
        return c

    prefix = lax.fori_loop(0, wid * (_TPW // _L), count_step,
                           jnp.zeros((_L,), jnp.int32))
    total = lax.fori_loop(wid * (_TPW // _L), _T // _L, count_step, prefix)

    pc = lax.shift_left(
        lax.shift_right_logical(total + (_TB - 1), _TB_LOG2), _TB_LOG2)
    cum = plsc.cumsum(pc)
    off = cum - pc

    @pl.when(wid == 0)
    def _():
        for half in range(_NTE // _L):
            tile_ids = (lane + half * _L) * _TB
            te = jnp.zeros((_L,), jnp.int32)
            for e in range(_E):
                bound = jnp.sum(jnp.where(lane == e, cum, 0))
                te = te + jnp.where(tile_ids >= bound, 1, 0)
            te_v[pl.ds(half * _L, _L)] = jnp.minimum(te, _E - 1)
        pltpu.sync_copy(te_v, te_hbm)
        nt = lax.shift_right_logical(
            jnp.sum(jnp.where(lane == _E - 1, cum, 0)), _TB_LOG2)
        nt_v[...] = jnp.zeros((_L,), jnp.int32) + nt
        pltpu.sync_copy(nt_v, nt_hbm)

    opr = off + prefix
    for j in range(_APW // _L):
        a_vec = wid * _APW + j * _L + lane
        toks = lax.shift_right_logical(a_vec, 1)
        kk = lax.bitwise_and(a_vec, 1)
        cvec = plsc.load_gather(codes_v, [toks])
        evec = jnp.where(kk == 0,
                         lax.shift_right_logical(cvec, 3),
                         lax.bitwise_and(cvec, 7))
        slot = jnp.zeros((_L,), jnp.int32)
        newopr = opr
        for e in range(_E):
            msk = evec == e
            cs = plsc.cumsum(jnp.where(msk, 1, 0))
            base_e = jnp.sum(jnp.where(lane == e, opr, 0))
            slot = jnp.where(msk, base_e + cs - 1, slot)
            cnt = jnp.sum(jnp.where(msk, 1, 0))
            newopr = jnp.where(lane == e, newopr + cnt, newopr)
        opr = newopr
        slots_v[pl.ds(j * _L, _L)] = slot
    pltpu.sync_copy(slots_v, slots_hbm.at[pl.ds(wid * _APW, _APW)])

    ssems = (ssem_a, ssem_b)
    sdesc = [None, None]
    for j in range(_APW // _L):
        p = j & 1
        if sdesc[p] is not None:
            sdesc[p].wait()
        a_vec = wid * _APW + j * _L + lane
        gidx_v[p, :] = lax.shift_right_logical(a_vec, 1)
        sidx_v[p, :] = slots_v[pl.ds(j * _L, _L)]
        pltpu.async_copy(x_hbm.at[gidx_v.at[p]], rows_v.at[p], gsem).wait()
        sdesc[p] = pltpu.async_copy(rows_v.at[p], dx_hbm.at[sidx_v.at[p]],
                                    ssems[p])
    sdesc[0].wait()
    sdesc[1].wait()


def _ffn_body(te_ref, dx_ref, w1_ref, b1_ref, w2_ref, b2_ref, yin_ref, y_ref):
    f = pl.program_id(0)
    h = jnp.dot(dx_ref[...], w1_ref[0], preferred_element_type=jnp.float32)
    h = jnp.maximum(h + b1_ref[0], 0.0)
    part = jnp.dot(h, w2_ref[0], preferred_element_type=jnp.float32)

    @pl.when(f == 0)
    def _():
        y_ref[...] = part + b2_ref[0]

    @pl.when(f == 1)
    def _():
        y_ref[...] = yin_ref[...] + part


@functools.partial(
    pl.kernel,
    out_type=jax.ShapeDtypeStruct((_T, _D), jnp.float32),
    mesh=_SC_MESH,
    scratch_types=[
        pltpu.VMEM((_APW,), jnp.int32),
        pltpu.VMEM((_TPW,), jnp.float32),
        pltpu.VMEM((2, _L), jnp.int32),
        pltpu.VMEM((2, _L, _D), jnp.float32),
        pltpu.VMEM((_L // 2, _D), jnp.float32),
        pltpu.SemaphoreType.DMA,
        pltpu.SemaphoreType.DMA,
    ],
    compiler_params=pltpu.CompilerParams(needs_layout_passes=False),
)
def _combine_kernel(y_hbm, slots_hbm, w1n_hbm, out_hbm,
                    slots_v, w_v, sidx_v, rows_v, ov, sem_a, sem_b):
    core = lax.axis_index("c")
    sub = lax.axis_index("s")
    wid = sub * _NC + core
    lane = lax.broadcasted_iota(jnp.int32, (_L,), 0)

    pltpu.sync_copy(slots_hbm.at[pl.ds(wid * _APW, _APW)], slots_v)
    pltpu.sync_copy(w1n_hbm.at[pl.ds(wid * _TPW, _TPW)], w_v)

    ntok = _L // 2
    ngrp = _TPW // ntok
    gsems = (sem_a, sem_b)
    gdesc = [None, None]
    sidx_v[0, :] = slots_v[pl.ds(0, _L)]
    gdesc[0] = pltpu.async_copy(y_hbm.at[sidx_v.at[0]], rows_v.at[0], gsems[0])
    for g in range(ngrp):
        p = g & 1
        if g + 1 < ngrp:
            sidx_v[1 - p, :] = slots_v[pl.ds((g + 1) * _L, _L)]
            gdesc[1 - p] = pltpu.async_copy(
                y_hbm.at[sidx_v.at[1 - p]], rows_v.at[1 - p], gsems[1 - p])
        gdesc[p].wait()
        wchunk = w_v[pl.ds((g // 2) * _L, _L)]
        wsplat = [
            jnp.sum(jnp.where(lane == (g % 2) * ntok + i, wchunk, 0.0))
            for i in range(ntok)
        ]

        def col_step(c, carry):
            for i in range(ntok):
                r0 = rows_v[p, 2 * i, pl.ds(c * _L, _L)]
                r1 = rows_v[p, 2 * i + 1, pl.ds(c * _L, _L)]
                ov[i, pl.ds(c * _L, _L)] = wsplat[i] * (r0 - r1) + r1
            return carry

        lax.fori_loop(0, _D // _L, col_step, jnp.int32(0))
        pltpu.sync_copy(ov, out_hbm.at[pl.ds(wid * _TPW + g * ntok, ntok)])


def kernel(x, gate_W, gate_b, W1, b1, W2, b2):
    code, w1n, loss = pl.pallas_call(
        _gating_body,
        out_shape=(
            jax.ShapeDtypeStruct((_T, 1), jnp.int32),
            jax.ShapeDtypeStruct((_T, 1), jnp.float32),
            jax.ShapeDtypeStruct((1, 1), jnp.float32),
        ),
    )(x, gate_W, gate_b.reshape(1, _E))

    dx, slots, te, ntv = _route_kernel(code.reshape(_T), x)

    grid_spec = pltpu.PrefetchScalarGridSpec(
        num_scalar_prefetch=1,
        grid=(2, ntv[0]),
        in_specs=[
            pl.BlockSpec((_TB, _D), lambda f, i, te: (i, 0)),
            pl.BlockSpec((1, _D, _FF // 2), lambda f, i, te: (te[i], 0, f)),
            pl.BlockSpec((1, 1, _FF // 2), lambda f, i, te: (te[i], 0, f)),
            pl.BlockSpec((1, _FF // 2, _D), lambda f, i, te: (te[i], f, 0)),
            pl.BlockSpec((1, 1, _D), lambda f, i, te: (te[i], 0, 0)),
            pl.BlockSpec((_TB, _D), lambda f, i, te: (i, 0)),
        ],
        out_specs=pl.BlockSpec((_TB, _D), lambda f, i, te: (i, 0)),
    )
    y = pl.pallas_call(
        _ffn_body,
        grid_spec=grid_spec,
        out_shape=jax.ShapeDtypeStruct((_SP, _D), jnp.float32),
        input_output_aliases={6: 0},
        compiler_params=pltpu.CompilerParams(
            dimension_semantics=("arbitrary", "arbitrary"),
        ),
    )(
        te,
        dx,
        W1,
        b1.reshape(_E, 1, _FF),
        W2,
        b2.reshape(_E, 1, _D),
        jnp.zeros((_SP, _D), jnp.float32),
    )

    out = _combine_kernel(y, slots, w1n.reshape(_T))
    return out, loss[0, 0]


# speedup vs baseline: 1.2122x; 1.0576x over previous
import functools

import jax
import jax.numpy as jnp
from jax import lax
from jax.experimental import pallas as pl
from jax.experimental.pallas import tpu as pltpu
from jax.experimental.pallas import tpu_sc as plsc

_E = 8
_K = 2
_D = 1024
_FF = 4096
_T = 2048

_NC = 2
_NS = 16
_NW = _NC * _NS
_L = 16

_TB = 256
_TB_LOG2 = 8
_NT = _T * _K // _TB + _E
_SP = _NT * _TB
_NTE = 48
_APW = _T * _K // _NW
_TPW = _T // _NW


def _gating_body(x_ref, gw_ref, gb_ref, code_ref, w1n_ref, loss_ref):
    x = x_ref[...]
    logits = jnp.dot(x, gw_ref[...], preferred_element_type=jnp.float32)
    logits = logits + gb_ref[...]
    m = jnp.max(logits, axis=-1, keepdims=True)
    ex = jnp.exp(logits - m)
    probs = ex / jnp.sum(ex, axis=-1, keepdims=True)
    eidx = lax.broadcasted_iota(jnp.int32, probs.shape, 1)
    m1 = jnp.max(probs, axis=-1, keepdims=True)
    i1 = jnp.min(jnp.where(probs == m1, eidx, _E), axis=-1, keepdims=True)
    probs2 = jnp.where(eidx == i1, -1.0, probs)
    m2 = jnp.max(probs2, axis=-1, keepdims=True)
    i2 = jnp.min(jnp.where(probs2 == m2, eidx, _E), axis=-1, keepdims=True)
    code_ref[...] = i1 * _E + i2
    w1n_ref[...] = m1 / (m1 + m2)
    oh = ((eidx == i1) | (eidx == i2)).astype(jnp.float32)
    imp = jnp.sum(probs, axis=0, keepdims=True)
    imp = imp / jnp.sum(imp)
    load = jnp.sum(oh, axis=0, keepdims=True) / (_T * _K)
    loss_ref[...] = _E * jnp.sum(imp * load, axis=(0, 1), keepdims=True)


_SC_MESH = plsc.VectorSubcoreMesh(
    core_axis_name="c", subcore_axis_name="s", num_cores=_NC, num_subcores=_NS
)


@functools.partial(
    pl.kernel,
    out_type=(
        jax.ShapeDtypeStruct((_SP, _D), jnp.float32),
        jax.ShapeDtypeStruct((_T * _K,), jnp.int32),
        jax.ShapeDtypeStruct((_NTE,), jnp.int32),
        jax.ShapeDtypeStruct((_L,), jnp.int32),
    ),
    mesh=_SC_MESH,
    scratch_types=[
        pltpu.VMEM((_T,), jnp.int32),
        pltpu.VMEM((_APW,), jnp.int32),
        pltpu.VMEM((2, _L), jnp.int32),
        pltpu.VMEM((2, _L), jnp.int32),
        pltpu.VMEM((2, _L, _D), jnp.float32),
        pltpu.VMEM((_NTE,), jnp.int32),
        pltpu.VMEM((_L,), jnp.int32),
        pltpu.SemaphoreType.DMA,
        pltpu.SemaphoreType.DMA,
        pltpu.SemaphoreType.DMA,
    ],
    compiler_params=pltpu.CompilerParams(needs_layout_passes=False),
)
def _route_kernel(code_hbm, x_hbm, dx_hbm, slots_hbm, te_hbm, nt_hbm,
                  codes_v, slots_v, gidx_v, sidx_v, rows_v, te_v, nt_v,
                  gsem, ssem_a, ssem_b):
    core = lax.axis_index("c")
    sub = lax.axis_index("s")
    wid = sub * _NC + core
    lane = lax.broadcasted_iota(jnp.int32, (_L,), 0)

    pltpu.sync_copy(code_hbm, codes_v)

    def count_step(j, c):
        v = codes_v[pl.ds(j * _L, _L)]
        e1 = lax.shift_right_logical(v, 3)
        e2 = lax.bitwise_and(v, 7)
        for e in range(_E):
            cnt = jnp.sum(jnp.where(e1 == e, 1, 0) + jnp.where(e2 == e, 1, 0))
            c = jnp.where(lane == e, c + cnt, c)
        return c

    prefix = lax.fori_loop(0, wid * (_TPW // _L), count_step,
                           jnp.zeros((_L,), jnp.int32))
    total = lax.fori_loop(wid * (_TPW // _L), _T // _L, count_step, prefix)

    pc = lax.shift_left(
        lax.shift_right_logical(total + (_TB - 1), _TB_LOG2), _TB_LOG2)
    cum = plsc.cumsum(pc)
    off = cum - pc

    @pl.when(wid == 0)
    def _():
        for half in range(_NTE // _L):
            tile_ids = (lane + half * _L) * _TB
            te = jnp.zeros((_L,), jnp.int32)
            for e in range(_E):
                bound = jnp.sum(jnp.where(lane == e, cum, 0))
                te = te + jnp.where(tile_ids >= bound, 1, 0)
            te_v[pl.ds(half * _L, _L)] = jnp.minimum(te, _E - 1)
        pltpu.sync_copy(te_v, te_hbm)
        nt = lax.shift_right_logical(
            jnp.sum(jnp.where(lane == _E - 1, cum, 0)), _TB_LOG2)
        nt_v[...] = jnp.zeros((_L,), jnp.int32) + nt
        pltpu.sync_copy(nt_v, nt_hbm)

    opr = off + prefix
    for j in range(_APW // _L):
        a_vec = wid * _APW + j * _L + lane
        toks = lax.shift_right_logical(a_vec, 1)
        kk = lax.bitwise_and(a_vec, 1)
        cvec = plsc.load_gather(codes_v, [toks])
        evec = jnp.where(kk == 0,
                         lax.shift_right_logical(cvec, 3),
                         lax.bitwise_and(cvec, 7))
        slot = jnp.zeros((_L,), jnp.int32)
        newopr = opr
        for e in range(_E):
            msk = evec == e
            cs = plsc.cumsum(jnp.where(msk, 1, 0))
            base_e = jnp.sum(jnp.where(lane == e, opr, 0))
            slot = jnp.where(msk, base_e + cs - 1, slot)
            cnt = jnp.sum(jnp.where(msk, 1, 0))
            newopr = jnp.where(lane == e, newopr + cnt, newopr)
        opr = newopr
        slots_v[pl.ds(j * _L, _L)] = slot
    pltpu.sync_copy(slots_v, slots_hbm.at[pl.ds(wid * _APW, _APW)])

    ssems = (ssem_a, ssem_b)
    sdesc = [None, None]
    for j in range(_APW // _L):
        p = j & 1
        if sdesc[p] is not None:
            sdesc[p].wait()
        a_vec = wid * _APW + j * _L + lane
        gidx_v[p, :] = lax.shift_right_logical(a_vec, 1)
        sidx_v[p, :] = slots_v[pl.ds(j * _L, _L)]
        pltpu.async_copy(x_hbm.at[gidx_v.at[p]], rows_v.at[p], gsem).wait()
        sdesc[p] = pltpu.async_copy(rows_v.at[p], dx_hbm.at[sidx_v.at[p]],
                                    ssems[p])
    sdesc[0].wait()
    sdesc[1].wait()


def _ffn_body(te_ref, dx_ref, w1_ref, b1_ref, w2_ref, b2_ref, y_ref, acc_ref):
    f = pl.program_id(0)
    i = pl.program_id(1)
    h = jnp.dot(dx_ref[...], w1_ref[0], preferred_element_type=jnp.float32)
    h = jnp.maximum(h + b1_ref[0], 0.0)
    part = jnp.dot(h, w2_ref[0], preferred_element_type=jnp.float32)

    @pl.when(f == 0)
    def _():
        acc_ref[pl.ds(i * _TB, _TB), :] = (part + b2_ref[0]).astype(jnp.bfloat16)

    @pl.when(f == 1)
    def _():
        y_ref[...] = acc_ref[pl.ds(i * _TB, _TB), :].astype(jnp.float32) + part


@functools.partial(
    pl.kernel,
    out_type=jax.ShapeDtypeStruct((_T, _D), jnp.float32),
    mesh=_SC_MESH,
    scratch_types=[
        pltpu.VMEM((_APW,), jnp.int32),
        pltpu.VMEM((_TPW,), jnp.float32),
        pltpu.VMEM((2, _L), jnp.int32),
        pltpu.VMEM((2, _L, _D), jnp.float32),
        pltpu.VMEM((_L // 2, _D), jnp.float32),
        pltpu.SemaphoreType.DMA,
        pltpu.SemaphoreType.DMA,
    ],
    compiler_params=pltpu.CompilerParams(needs_layout_passes=False),
)
def _combine_kernel(y_hbm, slots_hbm, w1n_hbm, out_hbm,
                    slots_v, w_v, sidx_v, rows_v, ov, sem_a, sem_b):
    core = lax.axis_index("c")
    sub = lax.axis_index("s")
    wid = sub * _NC + core
    lane = lax.broadcasted_iota(jnp.int32, (_L,), 0)

    pltpu.sync_copy(slots_hbm.at[pl.ds(wid * _APW, _APW)], slots_v)
    pltpu.sync_copy(w1n_hbm.at[pl.ds(wid * _TPW, _TPW)], w_v)

    ntok = _L // 2
    ngrp = _TPW // ntok
    gsems = (sem_a, sem_b)
    gdesc = [None, None]
    sidx_v[0, :] = slots_v[pl.ds(0, _L)]
    gdesc[0] = pltpu.async_copy(y_hbm.at[sidx_v.at[0]], rows_v.at[0], gsems[0])
    for g in range(ngrp):
        p = g & 1
        if g + 1 < ngrp:
            sidx_v[1 - p, :] = slots_v[pl.ds((g + 1) * _L, _L)]
            gdesc[1 - p] = pltpu.async_copy(
                y_hbm.at[sidx_v.at[1 - p]], rows_v.at[1 - p], gsems[1 - p])
        gdesc[p].wait()
        wchunk = w_v[pl.ds((g // 2) * _L, _L)]
        wsplat = [
            jnp.sum(jnp.where(lane == (g % 2) * ntok + i, wchunk, 0.0))
            for i in range(ntok)
        ]

        def col_step(c, carry):
            for i in range(ntok):
                r0 = rows_v[p, 2 * i, pl.ds(c * _L, _L)]
                r1 = rows_v[p, 2 * i + 1, pl.ds(c * _L, _L)]
                ov[i, pl.ds(c * _L, _L)] = wsplat[i] * (r0 - r1) + r1
            return carry

        lax.fori_loop(0, _D // _L, col_step, jnp.int32(0))
        pltpu.sync_copy(ov, out_hbm.at[pl.ds(wid * _TPW + g * ntok, ntok)])


def kernel(x, gate_W, gate_b, W1, b1, W2, b2):
    code, w1n, loss = pl.pallas_call(
        _gating_body,
        out_shape=(
            jax.ShapeDtypeStruct((_T, 1), jnp.int32),
            jax.ShapeDtypeStruct((_T, 1), jnp.float32),
            jax.ShapeDtypeStruct((1, 1), jnp.float32),
        ),
    )(x, gate_W, gate_b.reshape(1, _E))

    dx, slots, te, ntv = _route_kernel(code.reshape(_T), x)

    grid_spec = pltpu.PrefetchScalarGridSpec(
        num_scalar_prefetch=1,
        grid=(2, ntv[0]),
        in_specs=[
            pl.BlockSpec((_TB, _D), lambda f, i, te: (i, 0)),
            pl.BlockSpec((1, _D, _FF // 2), lambda f, i, te: (te[i], 0, f)),
            pl.BlockSpec((1, 1, _FF // 2), lambda f, i, te: (te[i], 0, f)),
            pl.BlockSpec((1, _FF // 2, _D), lambda f, i, te: (te[i], f, 0)),
            pl.BlockSpec((1, 1, _D), lambda f, i, te: (te[i], 0, 0)),
        ],
        out_specs=pl.BlockSpec((_TB, _D), lambda f, i, te: (i, 0)),
        scratch_shapes=[pltpu.VMEM((_SP, _D), jnp.bfloat16)],
    )
    y = pl.pallas_call(
        _ffn_body,
        grid_spec=grid_spec,
        out_shape=jax.ShapeDtypeStruct((_SP, _D), jnp.float32),
        compiler_params=pltpu.CompilerParams(
            dimension_semantics=("arbitrary", "arbitrary"),
        ),
    )(
        te,
        dx,
        W1,
        b1.reshape(_E, 1, _FF),
        W2,
        b2.reshape(_E, 1, _D),
    )

    out = _combine_kernel(y, slots, w1n.reshape(_T))
    return out, loss[0, 0]
